# Initial kernel scaffold; baseline (speedup 1.0000x reference)
import jax
import jax.numpy as jnp
from jax.experimental import pallas as pl


def kernel(feat, coord, Wq, bq, Wk, bk, Wv, bv, Wp1, bp1, Wp2, bp2, Ww1, bw1, Ww2, bw2, Wf1, bf1, Wf3, bf3):
    raise NotImplementedError("write your pallas kernel here")


# speedup vs baseline: 4.5963x; 4.5963x over previous
import functools

import jax
import jax.numpy as jnp
from jax import lax
from jax.experimental import pallas as pl
from jax.experimental.pallas import tpu as pltpu
from jax.experimental.pallas import tpu_sc as plsc

N = 8192
C = 256
G = 8
K = 16
NK = N * K
EPS = 1e-5
F32 = jnp.float32


def _nrm(y, stats, n):
    m = stats[0:1, :] / n
    var = stats[1:2, :] / n - m * m
    return (y - m) * lax.rsqrt(var + EPS)


def _acc_stats(ref, y, first):
    @pl.when(first)
    def _():
        ref[...] = jnp.zeros_like(ref)

    ref[0:1, :] += jnp.sum(y, axis=0, keepdims=True)
    ref[1:2, :] += jnp.sum(y * y, axis=0, keepdims=True)


_KNN_B = 128


def _knn_body(crd_ref, crdall_ref, idx_ref):
    blk = crd_ref[...]
    allc = crdall_ref[...]
    dot = lax.dot_general(blk, allc, (((1,), (1,)), ((), ())),
                          preferred_element_type=F32)
    rowd2 = jnp.sum(blk * blk, axis=1, keepdims=True)
    sq = allc * allc
    cold2 = lax.dot_general(jnp.ones((8, 8), F32), sq,
                            (((1,), (1,)), ((), ())),
                            preferred_element_type=F32,
                            precision=lax.Precision.HIGHEST)[0:1, :]
    d = rowd2 + cold2 - 2.0 * dot
    cols = lax.broadcasted_iota(jnp.int32, (_KNN_B, N), 1)
    for t in range(K):
        m = jnp.min(d, axis=1, keepdims=True)
        am = jnp.min(jnp.where(d <= m, cols, N), axis=1, keepdims=True)
        idx_ref[:, t:t + 1] = am
        d = jnp.where(cols == am, jnp.float32(3.0e38), d)


def _knn(coord):
    coordp = jnp.pad(coord, ((0, 0), (0, 5)))
    return pl.pallas_call(
        _knn_body,
        grid=(N // _KNN_B,),
        in_specs=[
            pl.BlockSpec((_KNN_B, 8), lambda i: (i, 0)),
            pl.BlockSpec((N, 8), lambda i: (0, 0)),
        ],
        out_specs=pl.BlockSpec((_KNN_B, K), lambda i: (i, 0)),
        out_shape=jax.ShapeDtypeStruct((N, K), jnp.int32),
    )(coordp, coordp)


def _sc_pos(cx_h, cy_h, cz_h, fidx):
    mesh = plsc.VectorSubcoreMesh(core_axis_name="c", subcore_axis_name="s")
    nper = N // 32

    @functools.partial(
        pl.kernel, mesh=mesh,
        out_type=jax.ShapeDtypeStruct((NK * 8,), F32),
        compiler_params=pltpu.CompilerParams(needs_layout_passes=False),
        scratch_types=[
            pltpu.VMEM((N,), F32),
            pltpu.VMEM((N,), F32),
            pltpu.VMEM((N,), F32),
            pltpu.VMEM((nper * K,), jnp.int32),
            pltpu.VMEM((nper * K * 8,), F32),
        ],
    )
    def body(cx_hbm, cy_hbm, cz_hbm, fidx_hbm, out_hbm, cx, cy, cz, idxv,
             posb):
        wid = lax.axis_index("s") * 2 + lax.axis_index("c")
        base = wid * nper
        pltpu.sync_copy(cx_hbm, cx)
        pltpu.sync_copy(cy_hbm, cy)
        pltpu.sync_copy(cz_hbm, cz)
        pltpu.sync_copy(fidx_hbm.at[pl.ds(base * K, nper * K)], idxv)
        lanes = lax.broadcasted_iota(jnp.int32, (16,), 0)
        zero = jnp.zeros((16,), F32)

        def step(n, _):
            iv = idxv[pl.ds(n * K, K)]
            ctr_i = jnp.full((16,), base + n, jnp.int32)
            flat = jnp.full((16,), n * K * 8, jnp.int32) + lanes * 8
            for c, cref in ((0, cx), (1, cy), (2, cz)):
                g = plsc.load_gather(cref, [iv])
                ctr = plsc.load_gather(cref, [ctr_i])
                plsc.store_scatter(posb, [flat + c], g - ctr)
            for c in (3, 4, 5, 6, 7):
                plsc.store_scatter(posb, [flat + c], zero)
            return 0

        lax.fori_loop(0, nper, step, 0)
        pltpu.sync_copy(posb, out_hbm.at[pl.ds(base * K * 8, nper * K * 8)])

    return body(cx_h, cy_h, cz_h, fidx)


def _sc_gather8(tab_flat, fidx):
    mesh = plsc.VectorSubcoreMesh(core_axis_name="c", subcore_axis_name="s")
    rper = NK // 32

    @functools.partial(
        pl.kernel, mesh=mesh,
        out_type=jax.ShapeDtypeStruct((NK * 8,), F32),
        compiler_params=pltpu.CompilerParams(needs_layout_passes=False),
        scratch_types=[
            pltpu.VMEM((N * 8,), F32),
            pltpu.VMEM((rper,), jnp.int32),
            pltpu.VMEM((rper * 8,), F32),
        ],
    )
    def body(tab_hbm, fidx_hbm, out_hbm, tabv, idxv, outb):
        wid = lax.axis_index("s") * 2 + lax.axis_index("c")
        base = wid * rper
        pltpu.sync_copy(tab_hbm, tabv)
        pltpu.sync_copy(fidx_hbm.at[pl.ds(base, rper)], idxv)
        lanes = lax.broadcasted_iota(jnp.int32, (16,), 0)

        def step(r, _):
            iv = idxv[pl.ds(r * 16, 16)] * 8
            flat = jnp.full((16,), r * 16 * 8, jnp.int32) + lanes * 8
            for c in range(8):
                g = plsc.load_gather(tabv, [iv + c])
                plsc.store_scatter(outb, [flat + c], g)
            return 0

        lax.fori_loop(0, rper // 16, step, 0)
        pltpu.sync_copy(outb, out_hbm.at[pl.ds(base * 8, rper * 8)])

    return body(tab_flat, fidx)


def _sc_gather(table, fidx, dcols):
    mesh = plsc.VectorSubcoreMesh(core_axis_name="c", subcore_axis_name="s")
    rper = NK // 32
    chunk = 128
    nch = rper // chunk

    @functools.partial(
        pl.kernel, mesh=mesh,
        out_type=jax.ShapeDtypeStruct((NK, dcols), F32),
        scratch_types=[
            pltpu.VMEM((rper,), jnp.int32),
            pltpu.VMEM((chunk, dcols), F32),
            pltpu.VMEM((chunk, dcols), F32),
            pltpu.SemaphoreType.DMA,
            pltpu.SemaphoreType.DMA,
        ],
    )
    def body(tab_hbm, fidx_hbm, out_hbm, idxv, b0, b1, s0, s1):
        wid = lax.axis_index("s") * 2 + lax.axis_index("c")
        base = wid * rper
        pltpu.sync_copy(fidx_hbm.at[pl.ds(base, rper)], idxv)
        bufs = (b0, b1)
        sems = (s0, s1)
        handles = [None] * nch
        handles[0] = pltpu.async_copy(
            tab_hbm.at[idxv.at[pl.ds(0, chunk)]], b0, s0)
        for c in range(nch):
            if c + 1 < nch:
                handles[c + 1] = pltpu.async_copy(
                    tab_hbm.at[idxv.at[pl.ds((c + 1) * chunk, chunk)]],
                    bufs[(c + 1) % 2], sems[(c + 1) % 2])
            handles[c].wait()
            pltpu.sync_copy(bufs[c % 2],
                            out_hbm.at[pl.ds(base + c * chunk, chunk)])

    return body(table, fidx)


_BM = 512


def _mm_stats_body(x_ref, w_ref, b_ref, y_ref, st_ref):
    y = jnp.dot(x_ref[...], w_ref[...], preferred_element_type=F32) \
        + b_ref[0:1, :]
    y_ref[...] = y
    _acc_stats(st_ref, y, pl.program_id(0) == 0)


def _mm_stats(x, w, b):
    return pl.pallas_call(
        _mm_stats_body,
        grid=(N // _BM,),
        in_specs=[
            pl.BlockSpec((_BM, C), lambda i: (i, 0)),
            pl.BlockSpec((C, C), lambda i: (0, 0)),
            pl.BlockSpec((8, C), lambda i: (0, 0)),
        ],
        out_specs=[
            pl.BlockSpec((_BM, C), lambda i: (i, 0)),
            pl.BlockSpec((8, C), lambda i: (0, 0)),
        ],
        out_shape=[
            jax.ShapeDtypeStruct((N, C), F32),
            jax.ShapeDtypeStruct((8, C), F32),
        ],
    )(x, w, b)


def _fqkv_body(y1_ref, s1_ref, wq_ref, bq_ref, wk_ref, bk_ref, wv_ref, bv_ref,
               yq_ref, yk_ref, v_ref, sq_ref, sk_ref):
    f = jax.nn.relu(_nrm(y1_ref[...], s1_ref, float(N)))
    first = pl.program_id(0) == 0
    yq = jnp.dot(f, wq_ref[...], preferred_element_type=F32) + bq_ref[0:1, :]
    yq_ref[...] = yq
    _acc_stats(sq_ref, yq, first)
    yk = jnp.dot(f, wk_ref[...], preferred_element_type=F32) + bk_ref[0:1, :]
    yk_ref[...] = yk
    _acc_stats(sk_ref, yk, first)
    v_ref[...] = jnp.dot(f, wv_ref[...], preferred_element_type=F32) \
        + bv_ref[0:1, :]


def _fqkv(y1, s1, wq, bq, wk, bk, wv, bv):
    wspec = pl.BlockSpec((C, C), lambda i: (0, 0))
    bspec = pl.BlockSpec((8, C), lambda i: (0, 0))
    blk = pl.BlockSpec((_BM, C), lambda i: (i, 0))
    return pl.pallas_call(
        _fqkv_body,
        grid=(N // _BM,),
        in_specs=[blk, bspec, wspec, bspec, wspec, bspec, wspec, bspec],
        out_specs=[blk, blk, blk, bspec, bspec],
        out_shape=[
            jax.ShapeDtypeStruct((N, C), F32),
            jax.ShapeDtypeStruct((N, C), F32),
            jax.ShapeDtypeStruct((N, C), F32),
            jax.ShapeDtypeStruct((8, C), F32),
            jax.ShapeDtypeStruct((8, C), F32),
        ],
    )(y1, s1, wq, bq, wk, bk, wv, bv)


def _qkproj_body(yq_ref, sq_ref, yk_ref, sk_ref, w1_ref,
                 a_ref, bq_out_ref):
    q = jax.nn.relu(_nrm(yq_ref[...], sq_ref, float(N)))
    kf = jax.nn.relu(_nrm(yk_ref[...], sk_ref, float(N)))
    w1 = w1_ref[...]
    a_ref[...] = jnp.dot(kf, w1, preferred_element_type=F32)
    bq_out_ref[...] = jnp.dot(q, w1, preferred_element_type=F32)


def _qkproj(yq, sq, yk, sk, w1):
    blk = pl.BlockSpec((_BM, C), lambda i: (i, 0))
    oblk = pl.BlockSpec((_BM, G), lambda i: (i, 0))
    return pl.pallas_call(
        _qkproj_body,
        grid=(N // _BM,),
        in_specs=[blk, pl.BlockSpec((8, C), lambda i: (0, 0)), blk,
                  pl.BlockSpec((8, C), lambda i: (0, 0)),
                  pl.BlockSpec((C, G), lambda i: (0, 0))],
        out_specs=[oblk, oblk],
        out_shape=[
            jax.ShapeDtypeStruct((N, G), F32),
            jax.ShapeDtypeStruct((N, G), F32),
        ],
    )(yq, sq, yk, sk, w1)


_BP = 2048


def _gp_stats_body(pos_ref, wp1_ref, bp1_ref, st_ref):
    gp = jnp.dot(pos_ref[...], wp1_ref[...], preferred_element_type=F32) \
        + bp1_ref[0:1, :]
    _acc_stats(st_ref, gp, pl.program_id(0) == 0)


def _gp_stats(posmat, wp1p, bp1):
    return pl.pallas_call(
        _gp_stats_body,
        grid=(NK // _BP,),
        in_specs=[
            pl.BlockSpec((_BP, 8), lambda i: (i, 0)),
            pl.BlockSpec((8, C), lambda i: (0, 0)),
            pl.BlockSpec((8, C), lambda i: (0, 0)),
        ],
        out_specs=pl.BlockSpec((8, C), lambda i: (0, 0)),
        out_shape=jax.ShapeDtypeStruct((8, C), F32),
    )(posmat, wp1p, bp1)


def _wpre_body(pos_ref, wp1_ref, bp1_ref, s3_ref, ag_ref, bq_ref,
               wp2_ref, w1_ref, bp2_ref, bw1_ref, wpre_ref, sw_ref):
    gp = jnp.dot(pos_ref[...], wp1_ref[...], preferred_element_type=F32) \
        + bp1_ref[0:1, :]
    r = jax.nn.relu(_nrm(gp, s3_ref, float(NK)))
    w1 = w1_ref[...]
    wc = jnp.dot(wp2_ref[...], w1, preferred_element_type=F32)
    cw = jnp.dot(bp2_ref[0:1, :], w1, preferred_element_type=F32) \
        + bw1_ref[0:1, :]
    nb = _BP // K
    u = (lax.broadcasted_iota(jnp.int32, (_BP, nb), 0) // K
         == lax.broadcasted_iota(jnp.int32, (_BP, nb), 1)).astype(F32)
    bq_rep = jnp.dot(u, bq_ref[...], preferred_element_type=F32)
    wpre = ag_ref[...] - bq_rep + jnp.dot(r, wc, preferred_element_type=F32) \
        + cw
    wpre_ref[...] = wpre
    _acc_stats(sw_ref, wpre, pl.program_id(0) == 0)


def _wpre(posmat, wp1p, bp1, s3, ag, bq8, wp2, w1, bp2, bw1):
    cspec = pl.BlockSpec((8, C), lambda i: (0, 0))
    return pl.pallas_call(
        _wpre_body,
        grid=(NK // _BP,),
        in_specs=[
            pl.BlockSpec((_BP, 8), lambda i: (i, 0)),
            cspec, cspec, cspec,
            pl.BlockSpec((_BP, G), lambda i: (i, 0)),
            pl.BlockSpec((_BP // K, G), lambda i: (i, 0)),
            pl.BlockSpec((C, C), lambda i: (0, 0)),
            pl.BlockSpec((C, G), lambda i: (0, 0)),
            cspec,
            pl.BlockSpec((8, G), lambda i: (0, 0)),
        ],
        out_specs=[
            pl.BlockSpec((_BP, G), lambda i: (i, 0)),
            pl.BlockSpec((8, G), lambda i: (0, 0)),
        ],
        out_shape=[
            jax.ShapeDtypeStruct((NK, G), F32),
            jax.ShapeDtypeStruct((8, G), F32),
        ],
    )(posmat, wp1p, bp1, s3, ag, bq8, wp2, w1, bp2, bw1)


def _att_body(wpre_ref, sw_ref, ww2_ref, bw2_ref, pos_ref, wp1_ref, bp1_ref,
              s3_ref, wp2_ref, bp2_ref, vg_ref, wf3_ref, bf3_ref, e16_ref,
              z_ref, sz_ref):
    nb = _BP // K
    s2 = jax.nn.relu(_nrm(wpre_ref[...], sw_ref, float(NK)))
    wlin = jnp.dot(s2, ww2_ref[...], preferred_element_type=F32) \
        + bw2_ref[0:1, :]
    e = jnp.exp(wlin)
    sel = (lax.broadcasted_iota(jnp.int32, (nb, _BP), 0)
           == lax.broadcasted_iota(jnp.int32, (nb, _BP), 1) // K).astype(F32)
    denom = jnp.dot(sel, e, preferred_element_type=F32)
    u = (lax.broadcasted_iota(jnp.int32, (_BP, nb), 0) // K
         == lax.broadcasted_iota(jnp.int32, (_BP, nb), 1)).astype(F32)
    w = e * jnp.dot(u, 1.0 / denom, preferred_element_type=F32)
    wt = jnp.dot(w, e16_ref[...], preferred_element_type=F32)
    gp = jnp.dot(pos_ref[...], wp1_ref[...], preferred_element_type=F32) \
        + bp1_ref[0:1, :]
    r = jax.nn.relu(_nrm(gp, s3_ref, float(NK)))
    val = vg_ref[...] + jnp.dot(r, wp2_ref[...], preferred_element_type=F32) \
        + bp2_ref[0:1, :]
    att = jnp.dot(sel, wt * val, preferred_element_type=F32)
    z = jnp.dot(att, wf3_ref[...], preferred_element_type=F32) \
        + bf3_ref[0:1, :]
    z_ref[...] = z
    _acc_stats(sz_ref, z, pl.program_id(0) == 0)


def _att(wpre, sw, ww2, bw2, posmat, wp1p, bp1, s3, wp2, bp2, vg, wf3, bf3,
         e8):
    cspec = pl.BlockSpec((8, C), lambda i: (0, 0))
    gspec = pl.BlockSpec((8, G), lambda i: (0, 0))
    return pl.pallas_call(
        _att_body,
        grid=(NK // _BP,),
        in_specs=[
            pl.BlockSpec((_BP, G), lambda i: (i, 0)),
            gspec,
            pl.BlockSpec((G, G), lambda i: (0, 0)),
            gspec,
            pl.BlockSpec((_BP, 8), lambda i: (i, 0)),
            cspec, cspec, cspec,
            pl.BlockSpec((C, C), lambda i: (0, 0)),
            cspec,
            pl.BlockSpec((_BP, C), lambda i: (i, 0)),
            pl.BlockSpec((C, C), lambda i: (0, 0)),
            cspec,
            pl.BlockSpec((G, C), lambda i: (0, 0)),
        ],
        out_specs=[
            pl.BlockSpec((_BP // K, C), lambda i: (i, 0)),
            cspec,
        ],
        out_shape=[
            jax.ShapeDtypeStruct((N, C), F32),
            jax.ShapeDtypeStruct((8, C), F32),
        ],
    )(wpre, sw, ww2, bw2, posmat, wp1p, bp1, s3, wp2, bp2, vg, wf3, bf3,
      e8)


def _resid_body(x_ref, z_ref, sz_ref, out_ref):
    out_ref[...] = jax.nn.relu(x_ref[...] + _nrm(z_ref[...], sz_ref, float(N)))


def _resid(x, z, sz):
    blk = pl.BlockSpec((_BM, C), lambda i: (i, 0))
    return pl.pallas_call(
        _resid_body,
        grid=(N // _BM,),
        in_specs=[blk, blk, pl.BlockSpec((8, C), lambda i: (0, 0))],
        out_specs=blk,
        out_shape=jax.ShapeDtypeStruct((N, C), F32),
    )(x, z, sz)


def kernel(feat, coord, Wq, bq, Wk, bk, Wv, bv, Wp1, bp1, Wp2, bp2,
           Ww1, bw1, Ww2, bw2, Wf1, bf1, Wf3, bf3):
    D = Wq.shape[0]
    idx = _knn(coord)
    fidx = idx.reshape(-1)
    posmat = _sc_pos(coord[:, 0], coord[:, 1], coord[:, 2],
                     fidx).reshape(NK, 8)

    wp1p = jnp.concatenate([Wp1, jnp.zeros((D, 5, C), F32)], axis=1)
    e8 = (jnp.arange(G)[:, None]
          == (jnp.arange(C)[None, :] // (C // G))).astype(F32)

    def pad_b(b):
        return jnp.broadcast_to(b[None, :], (8, b.shape[0]))

    x = feat
    for i in range(D):
        y1, s1 = _mm_stats(x, Wf1[i], pad_b(bf1[i]))
        yq, yk, v, sq, sk = _fqkv(y1, s1, Wq[i], pad_b(bq[i]), Wk[i],
                                  pad_b(bk[i]), Wv[i], pad_b(bv[i]))
        a8, bq8 = _qkproj(yq, sq, yk, sk, Ww1[i])
        ag = _sc_gather8(a8.reshape(-1), fidx).reshape(NK, G)
        vg = _sc_gather(v, fidx, C)
        s3 = _gp_stats(posmat, wp1p[i], pad_b(bp1[i]))
        wpre, sw = _wpre(posmat, wp1p[i], pad_b(bp1[i]), s3, ag, bq8,
                         Wp2[i], Ww1[i], pad_b(bp2[i]), pad_b(bw1[i]))
        z, sz = _att(wpre, sw, Ww2[i], pad_b(bw2[i]), posmat, wp1p[i],
                     pad_b(bp1[i]), s3, Wp2[i], pad_b(bp2[i]), vg,
                     Wf3[i], pad_b(bf3[i]), e8)
        x = _resid(x, z, sz)
    return x


# speedup vs baseline: 6.2939x; 1.3693x over previous
import functools

import jax
import jax.numpy as jnp
from jax import lax
from jax.experimental import pallas as pl
from jax.experimental.pallas import tpu as pltpu
from jax.experimental.pallas import tpu_sc as plsc

N = 8192
C = 256
G = 8
K = 16
NK = N * K
EPS = 1e-5
F32 = jnp.float32


def _nrm(y, stats, n):
    m = stats[0:1, :] / n
    var = stats[1:2, :] / n - m * m
    return (y - m) * lax.rsqrt(var + EPS)


def _acc_stats(ref, y, first):
    @pl.when(first)
    def _():
        ref[...] = jnp.zeros_like(ref)

    ref[0:1, :] += jnp.sum(y, axis=0, keepdims=True)
    ref[1:2, :] += jnp.sum(y * y, axis=0, keepdims=True)


_KNN_B = 128


def _knn_body(crd_ref, crdall_ref, idx_ref):
    blk = crd_ref[...]
    allc = crdall_ref[...]
    dot = lax.dot_general(blk, allc, (((1,), (1,)), ((), ())),
                          preferred_element_type=F32)
    rowd2 = jnp.sum(blk * blk, axis=1, keepdims=True)
    sq = allc * allc
    cold2 = lax.dot_general(jnp.ones((8, 8), F32), sq,
                            (((1,), (1,)), ((), ())),
                            preferred_element_type=F32,
                            precision=lax.Precision.HIGHEST)[0:1, :]
    d = rowd2 + cold2 - 2.0 * dot
    INF = jnp.float32(3.0e38)
    BIGI = jnp.int32(N)
    lane = lax.broadcasted_iota(jnp.int32, (_KNN_B, 128), 1)
    ms = [jnp.full((_KNN_B, 128), INF) for _ in range(4)]
    js = [jnp.full((_KNN_B, 128), BIGI) for _ in range(4)]
    for j in range(64):
        e = d[:, j * 128:(j + 1) * 128]
        c = lane + (j * 128)
        for s in range(4):
            cond = (e < ms[s]) | ((e == ms[s]) & (c < js[s]))
            nm = jnp.where(cond, e, ms[s])
            nj = jnp.where(cond, c, js[s])
            e = jnp.where(cond, ms[s], e)
            c = jnp.where(cond, js[s], c)
            ms[s] = nm
            js[s] = nj
    v = jnp.concatenate(ms, axis=1)
    jc = jnp.concatenate(js, axis=1)
    pcnt = jnp.zeros((_KNN_B, 128), jnp.float32)
    flag = jnp.zeros((_KNN_B, 1), jnp.float32)
    for t in range(K):
        if t > 0:
            flag = flag + jnp.max(
                jnp.where(pcnt >= 4.0, 1.0, 0.0), axis=1, keepdims=True)
        vmin = jnp.min(v, axis=1, keepdims=True)
        jm = jnp.min(jnp.where(v <= vmin, jc, BIGI), axis=1, keepdims=True)
        idx_ref[:, t:t + 1] = jm
        v = jnp.where((v <= vmin) & (jc == jm), INF, v)
        pcnt = pcnt + jnp.where(lane == jm % 128, 1.0, 0.0)

    @pl.when(jnp.sum(flag) > 0.0)
    def _():
        dd = d
        cols = lax.broadcasted_iota(jnp.int32, (_KNN_B, N), 1)
        for t in range(K):
            m = jnp.min(dd, axis=1, keepdims=True)
            am = jnp.min(jnp.where(dd <= m, cols, BIGI), axis=1,
                         keepdims=True)
            idx_ref[:, t:t + 1] = am
            dd = jnp.where(cols == am, INF, dd)


def _knn(coord):
    coordp = jnp.pad(coord, ((0, 0), (0, 5)))
    return pl.pallas_call(
        _knn_body,
        grid=(N // _KNN_B,),
        in_specs=[
            pl.BlockSpec((_KNN_B, 8), lambda i: (i, 0)),
            pl.BlockSpec((N, 8), lambda i: (0, 0)),
        ],
        out_specs=pl.BlockSpec((_KNN_B, K), lambda i: (i, 0)),
        out_shape=jax.ShapeDtypeStruct((N, K), jnp.int32),
    )(coordp, coordp)


def _sc_pos(cx_h, cy_h, cz_h, fidx):
    mesh = plsc.VectorSubcoreMesh(core_axis_name="c", subcore_axis_name="s")
    nper = N // 32

    @functools.partial(
        pl.kernel, mesh=mesh,
        out_type=jax.ShapeDtypeStruct((NK * 8,), F32),
        compiler_params=pltpu.CompilerParams(needs_layout_passes=False),
        scratch_types=[
            pltpu.VMEM((N,), F32),
            pltpu.VMEM((N,), F32),
            pltpu.VMEM((N,), F32),
            pltpu.VMEM((nper * K,), jnp.int32),
            pltpu.VMEM((nper * K * 8,), F32),
        ],
    )
    def body(cx_hbm, cy_hbm, cz_hbm, fidx_hbm, out_hbm, cx, cy, cz, idxv,
             posb):
        wid = lax.axis_index("s") * 2 + lax.axis_index("c")
        base = wid * nper
        pltpu.sync_copy(cx_hbm, cx)
        pltpu.sync_copy(cy_hbm, cy)
        pltpu.sync_copy(cz_hbm, cz)
        pltpu.sync_copy(fidx_hbm.at[pl.ds(base * K, nper * K)], idxv)
        lanes = lax.broadcasted_iota(jnp.int32, (16,), 0)
        zero = jnp.zeros((16,), F32)

        def step(n, _):
            iv = idxv[pl.ds(n * K, K)]
            ctr_i = jnp.full((16,), base + n, jnp.int32)
            flat = jnp.full((16,), n * K * 8, jnp.int32) + lanes * 8
            for c, cref in ((0, cx), (1, cy), (2, cz)):
                g = plsc.load_gather(cref, [iv])
                ctr = plsc.load_gather(cref, [ctr_i])
                plsc.store_scatter(posb, [flat + c], g - ctr)
            for c in (3, 4, 5, 6, 7):
                plsc.store_scatter(posb, [flat + c], zero)
            return 0

        lax.fori_loop(0, nper, step, 0)
        pltpu.sync_copy(posb, out_hbm.at[pl.ds(base * K * 8, nper * K * 8)])

    return body(cx_h, cy_h, cz_h, fidx)


def _sc_gather8(tab_flat, fidx):
    mesh = plsc.VectorSubcoreMesh(core_axis_name="c", subcore_axis_name="s")
    rper = NK // 32

    @functools.partial(
        pl.kernel, mesh=mesh,
        out_type=jax.ShapeDtypeStruct((NK * 8,), F32),
        compiler_params=pltpu.CompilerParams(needs_layout_passes=False),
        scratch_types=[
            pltpu.VMEM((N * 8,), F32),
            pltpu.VMEM((rper,), jnp.int32),
            pltpu.VMEM((rper * 8,), F32),
        ],
    )
    def body(tab_hbm, fidx_hbm, out_hbm, tabv, idxv, outb):
        wid = lax.axis_index("s") * 2 + lax.axis_index("c")
        base = wid * rper
        pltpu.sync_copy(tab_hbm, tabv)
        pltpu.sync_copy(fidx_hbm.at[pl.ds(base, rper)], idxv)
        lanes = lax.broadcasted_iota(jnp.int32, (16,), 0)

        def step(r, _):
            iv = idxv[pl.ds(r * 16, 16)] * 8
            flat = jnp.full((16,), r * 16 * 8, jnp.int32) + lanes * 8
            for c in range(8):
                g = plsc.load_gather(tabv, [iv + c])
                plsc.store_scatter(outb, [flat + c], g)
            return 0

        lax.fori_loop(0, rper // 16, step, 0)
        pltpu.sync_copy(outb, out_hbm.at[pl.ds(base * 8, rper * 8)])

    return body(tab_flat, fidx)


def _sc_gather(table, fidx, dcols):
    mesh = plsc.VectorSubcoreMesh(core_axis_name="c", subcore_axis_name="s")
    rper = NK // 32
    chunk = 128
    nch = rper // chunk

    @functools.partial(
        pl.kernel, mesh=mesh,
        out_type=jax.ShapeDtypeStruct((NK, dcols), F32),
        scratch_types=[
            pltpu.VMEM((rper,), jnp.int32),
            pltpu.VMEM((chunk, dcols), F32),
            pltpu.VMEM((chunk, dcols), F32),
            pltpu.SemaphoreType.DMA,
            pltpu.SemaphoreType.DMA,
        ],
    )
    def body(tab_hbm, fidx_hbm, out_hbm, idxv, b0, b1, s0, s1):
        wid = lax.axis_index("s") * 2 + lax.axis_index("c")
        base = wid * rper
        pltpu.sync_copy(fidx_hbm.at[pl.ds(base, rper)], idxv)
        bufs = (b0, b1)
        sems = (s0, s1)
        handles = [None] * nch
        handles[0] = pltpu.async_copy(
            tab_hbm.at[idxv.at[pl.ds(0, chunk)]], b0, s0)
        for c in range(nch):
            if c + 1 < nch:
                handles[c + 1] = pltpu.async_copy(
                    tab_hbm.at[idxv.at[pl.ds((c + 1) * chunk, chunk)]],
                    bufs[(c + 1) % 2], sems[(c + 1) % 2])
            handles[c].wait()
            pltpu.sync_copy(bufs[c % 2],
                            out_hbm.at[pl.ds(base + c * chunk, chunk)])

    return body(table, fidx)


_BM = 512


def _mm_stats_body(x_ref, w_ref, b_ref, y_ref, st_ref):
    y = jnp.dot(x_ref[...], w_ref[...], preferred_element_type=F32) \
        + b_ref[0:1, :]
    y_ref[...] = y
    _acc_stats(st_ref, y, pl.program_id(0) == 0)


def _mm_stats(x, w, b):
    return pl.pallas_call(
        _mm_stats_body,
        grid=(N // _BM,),
        in_specs=[
            pl.BlockSpec((_BM, C), lambda i: (i, 0)),
            pl.BlockSpec((C, C), lambda i: (0, 0)),
            pl.BlockSpec((8, C), lambda i: (0, 0)),
        ],
        out_specs=[
            pl.BlockSpec((_BM, C), lambda i: (i, 0)),
            pl.BlockSpec((8, C), lambda i: (0, 0)),
        ],
        out_shape=[
            jax.ShapeDtypeStruct((N, C), F32),
            jax.ShapeDtypeStruct((8, C), F32),
        ],
    )(x, w, b)


def _fqkv_body(y1_ref, s1_ref, wq_ref, bq_ref, wk_ref, bk_ref, wv_ref, bv_ref,
               yq_ref, yk_ref, v_ref, sq_ref, sk_ref):
    f = jax.nn.relu(_nrm(y1_ref[...], s1_ref, float(N)))
    first = pl.program_id(0) == 0
    yq = jnp.dot(f, wq_ref[...], preferred_element_type=F32) + bq_ref[0:1, :]
    yq_ref[...] = yq
    _acc_stats(sq_ref, yq, first)
    yk = jnp.dot(f, wk_ref[...], preferred_element_type=F32) + bk_ref[0:1, :]
    yk_ref[...] = yk
    _acc_stats(sk_ref, yk, first)
    v_ref[...] = jnp.dot(f, wv_ref[...], preferred_element_type=F32) \
        + bv_ref[0:1, :]


def _fqkv(y1, s1, wq, bq, wk, bk, wv, bv):
    wspec = pl.BlockSpec((C, C), lambda i: (0, 0))
    bspec = pl.BlockSpec((8, C), lambda i: (0, 0))
    blk = pl.BlockSpec((_BM, C), lambda i: (i, 0))
    return pl.pallas_call(
        _fqkv_body,
        grid=(N // _BM,),
        in_specs=[blk, bspec, wspec, bspec, wspec, bspec, wspec, bspec],
        out_specs=[blk, blk, blk, bspec, bspec],
        out_shape=[
            jax.ShapeDtypeStruct((N, C), F32),
            jax.ShapeDtypeStruct((N, C), F32),
            jax.ShapeDtypeStruct((N, C), F32),
            jax.ShapeDtypeStruct((8, C), F32),
            jax.ShapeDtypeStruct((8, C), F32),
        ],
    )(y1, s1, wq, bq, wk, bk, wv, bv)


def _qkproj_body(yq_ref, sq_ref, yk_ref, sk_ref, w1_ref,
                 a_ref, bq_out_ref):
    q = jax.nn.relu(_nrm(yq_ref[...], sq_ref, float(N)))
    kf = jax.nn.relu(_nrm(yk_ref[...], sk_ref, float(N)))
    w1 = w1_ref[...]
    a_ref[...] = jnp.dot(kf, w1, preferred_element_type=F32)
    bq_out_ref[...] = jnp.dot(q, w1, preferred_element_type=F32)


def _qkproj(yq, sq, yk, sk, w1):
    blk = pl.BlockSpec((_BM, C), lambda i: (i, 0))
    oblk = pl.BlockSpec((_BM, G), lambda i: (i, 0))
    return pl.pallas_call(
        _qkproj_body,
        grid=(N // _BM,),
        in_specs=[blk, pl.BlockSpec((8, C), lambda i: (0, 0)), blk,
                  pl.BlockSpec((8, C), lambda i: (0, 0)),
                  pl.BlockSpec((C, G), lambda i: (0, 0))],
        out_specs=[oblk, oblk],
        out_shape=[
            jax.ShapeDtypeStruct((N, G), F32),
            jax.ShapeDtypeStruct((N, G), F32),
        ],
    )(yq, sq, yk, sk, w1)


_BP = 2048


def _gp_stats_body(pos_ref, wp1_ref, bp1_ref, st_ref):
    gp = jnp.dot(pos_ref[...], wp1_ref[...], preferred_element_type=F32) \
        + bp1_ref[0:1, :]
    _acc_stats(st_ref, gp, pl.program_id(0) == 0)


def _gp_stats(posmat, wp1p, bp1):
    return pl.pallas_call(
        _gp_stats_body,
        grid=(NK // _BP,),
        in_specs=[
            pl.BlockSpec((_BP, 8), lambda i: (i, 0)),
            pl.BlockSpec((8, C), lambda i: (0, 0)),
            pl.BlockSpec((8, C), lambda i: (0, 0)),
        ],
        out_specs=pl.BlockSpec((8, C), lambda i: (0, 0)),
        out_shape=jax.ShapeDtypeStruct((8, C), F32),
    )(posmat, wp1p, bp1)


def _wpre_body(pos_ref, wp1_ref, bp1_ref, s3_ref, ag_ref, bq_ref,
               wp2_ref, w1_ref, bp2_ref, bw1_ref, wpre_ref, sw_ref):
    gp = jnp.dot(pos_ref[...], wp1_ref[...], preferred_element_type=F32) \
        + bp1_ref[0:1, :]
    r = jax.nn.relu(_nrm(gp, s3_ref, float(NK)))
    w1 = w1_ref[...]
    wc = jnp.dot(wp2_ref[...], w1, preferred_element_type=F32)
    cw = jnp.dot(bp2_ref[0:1, :], w1, preferred_element_type=F32) \
        + bw1_ref[0:1, :]
    nb = _BP // K
    u = (lax.broadcasted_iota(jnp.int32, (_BP, nb), 0) // K
         == lax.broadcasted_iota(jnp.int32, (_BP, nb), 1)).astype(F32)
    bq_rep = jnp.dot(u, bq_ref[...], preferred_element_type=F32)
    wpre = ag_ref[...] - bq_rep + jnp.dot(r, wc, preferred_element_type=F32) \
        + cw
    wpre_ref[...] = wpre
    _acc_stats(sw_ref, wpre, pl.program_id(0) == 0)


def _wpre(posmat, wp1p, bp1, s3, ag, bq8, wp2, w1, bp2, bw1):
    cspec = pl.BlockSpec((8, C), lambda i: (0, 0))
    return pl.pallas_call(
        _wpre_body,
        grid=(NK // _BP,),
        in_specs=[
            pl.BlockSpec((_BP, 8), lambda i: (i, 0)),
            cspec, cspec, cspec,
            pl.BlockSpec((_BP, G), lambda i: (i, 0)),
            pl.BlockSpec((_BP // K, G), lambda i: (i, 0)),
            pl.BlockSpec((C, C), lambda i: (0, 0)),
            pl.BlockSpec((C, G), lambda i: (0, 0)),
            cspec,
            pl.BlockSpec((8, G), lambda i: (0, 0)),
        ],
        out_specs=[
            pl.BlockSpec((_BP, G), lambda i: (i, 0)),
            pl.BlockSpec((8, G), lambda i: (0, 0)),
        ],
        out_shape=[
            jax.ShapeDtypeStruct((NK, G), F32),
            jax.ShapeDtypeStruct((8, G), F32),
        ],
    )(posmat, wp1p, bp1, s3, ag, bq8, wp2, w1, bp2, bw1)


def _att_body(wpre_ref, sw_ref, ww2_ref, bw2_ref, pos_ref, wp1_ref, bp1_ref,
              s3_ref, wp2_ref, bp2_ref, vg_ref, wf3_ref, bf3_ref, e16_ref,
              z_ref, sz_ref):
    nb = _BP // K
    s2 = jax.nn.relu(_nrm(wpre_ref[...], sw_ref, float(NK)))
    wlin = jnp.dot(s2, ww2_ref[...], preferred_element_type=F32) \
        + bw2_ref[0:1, :]
    e = jnp.exp(wlin)
    sel = (lax.broadcasted_iota(jnp.int32, (nb, _BP), 0)
           == lax.broadcasted_iota(jnp.int32, (nb, _BP), 1) // K).astype(F32)
    denom = jnp.dot(sel, e, preferred_element_type=F32)
    u = (lax.broadcasted_iota(jnp.int32, (_BP, nb), 0) // K
         == lax.broadcasted_iota(jnp.int32, (_BP, nb), 1)).astype(F32)
    w = e * jnp.dot(u, 1.0 / denom, preferred_element_type=F32)
    wt = jnp.dot(w, e16_ref[...], preferred_element_type=F32)
    gp = jnp.dot(pos_ref[...], wp1_ref[...], preferred_element_type=F32) \
        + bp1_ref[0:1, :]
    r = jax.nn.relu(_nrm(gp, s3_ref, float(NK)))
    val = vg_ref[...] + jnp.dot(r, wp2_ref[...], preferred_element_type=F32) \
        + bp2_ref[0:1, :]
    att = jnp.dot(sel, wt * val, preferred_element_type=F32)
    z = jnp.dot(att, wf3_ref[...], preferred_element_type=F32) \
        + bf3_ref[0:1, :]
    z_ref[...] = z
    _acc_stats(sz_ref, z, pl.program_id(0) == 0)


def _att(wpre, sw, ww2, bw2, posmat, wp1p, bp1, s3, wp2, bp2, vg, wf3, bf3,
         e8):
    cspec = pl.BlockSpec((8, C), lambda i: (0, 0))
    gspec = pl.BlockSpec((8, G), lambda i: (0, 0))
    return pl.pallas_call(
        _att_body,
        grid=(NK // _BP,),
        in_specs=[
            pl.BlockSpec((_BP, G), lambda i: (i, 0)),
            gspec,
            pl.BlockSpec((G, G), lambda i: (0, 0)),
            gspec,
            pl.BlockSpec((_BP, 8), lambda i: (i, 0)),
            cspec, cspec, cspec,
            pl.BlockSpec((C, C), lambda i: (0, 0)),
            cspec,
            pl.BlockSpec((_BP, C), lambda i: (i, 0)),
            pl.BlockSpec((C, C), lambda i: (0, 0)),
            cspec,
            pl.BlockSpec((G, C), lambda i: (0, 0)),
        ],
        out_specs=[
            pl.BlockSpec((_BP // K, C), lambda i: (i, 0)),
            cspec,
        ],
        out_shape=[
            jax.ShapeDtypeStruct((N, C), F32),
            jax.ShapeDtypeStruct((8, C), F32),
        ],
    )(wpre, sw, ww2, bw2, posmat, wp1p, bp1, s3, wp2, bp2, vg, wf3, bf3,
      e8)


def _resid_body(x_ref, z_ref, sz_ref, out_ref):
    out_ref[...] = jax.nn.relu(x_ref[...] + _nrm(z_ref[...], sz_ref, float(N)))


def _resid(x, z, sz):
    blk = pl.BlockSpec((_BM, C), lambda i: (i, 0))
    return pl.pallas_call(
        _resid_body,
        grid=(N // _BM,),
        in_specs=[blk, blk, pl.BlockSpec((8, C), lambda i: (0, 0))],
        out_specs=blk,
        out_shape=jax.ShapeDtypeStruct((N, C), F32),
    )(x, z, sz)


def kernel(feat, coord, Wq, bq, Wk, bk, Wv, bv, Wp1, bp1, Wp2, bp2,
           Ww1, bw1, Ww2, bw2, Wf1, bf1, Wf3, bf3):
    D = Wq.shape[0]
    idx = _knn(coord)
    fidx = idx.reshape(-1)
    posmat = _sc_pos(coord[:, 0], coord[:, 1], coord[:, 2],
                     fidx).reshape(NK, 8)

    wp1p = jnp.concatenate([Wp1, jnp.zeros((D, 5, C), F32)], axis=1)
    e8 = (jnp.arange(G)[:, None]
          == (jnp.arange(C)[None, :] // (C // G))).astype(F32)

    def pad_b(b):
        return jnp.broadcast_to(b[None, :], (8, b.shape[0]))

    x = feat
    for i in range(D):
        y1, s1 = _mm_stats(x, Wf1[i], pad_b(bf1[i]))
        yq, yk, v, sq, sk = _fqkv(y1, s1, Wq[i], pad_b(bq[i]), Wk[i],
                                  pad_b(bk[i]), Wv[i], pad_b(bv[i]))
        a8, bq8 = _qkproj(yq, sq, yk, sk, Ww1[i])
        ag = _sc_gather8(a8.reshape(-1), fidx).reshape(NK, G)
        vg = _sc_gather(v, fidx, C)
        s3 = _gp_stats(posmat, wp1p[i], pad_b(bp1[i]))
        wpre, sw = _wpre(posmat, wp1p[i], pad_b(bp1[i]), s3, ag, bq8,
                         Wp2[i], Ww1[i], pad_b(bp2[i]), pad_b(bw1[i]))
        z, sz = _att(wpre, sw, Ww2[i], pad_b(bw2[i]), posmat, wp1p[i],
                     pad_b(bp1[i]), s3, Wp2[i], pad_b(bp2[i]), vg,
                     Wf3[i], pad_b(bf3[i]), e8)
        x = _resid(x, z, sz)
    return x


# speedup vs baseline: 7.3460x; 1.1672x over previous
import functools

import jax
import jax.numpy as jnp
from jax import lax
from jax.experimental import pallas as pl
from jax.experimental.pallas import tpu as pltpu
from jax.experimental.pallas import tpu_sc as plsc

N = 8192
C = 256
G = 8
K = 16
NK = N * K
EPS = 1e-5
F32 = jnp.float32


def _nrm(y, stats, n):
    m = stats[0:1, :] / n
    var = stats[1:2, :] / n - m * m
    return (y - m) * lax.rsqrt(var + EPS)


def _acc_stats(ref, y, first):
    @pl.when(first)
    def _():
        ref[...] = jnp.zeros_like(ref)

    ref[0:1, :] += jnp.sum(y, axis=0, keepdims=True)
    ref[1:2, :] += jnp.sum(y * y, axis=0, keepdims=True)


_KNN_B = 128


def _knn_body(crd_ref, crdall_ref, idx_ref):
    blk = crd_ref[...]
    allc = crdall_ref[...]
    dot = lax.dot_general(blk, allc, (((1,), (1,)), ((), ())),
                          preferred_element_type=F32)
    rowd2 = jnp.sum(blk * blk, axis=1, keepdims=True)
    sq = allc * allc
    cold2 = lax.dot_general(jnp.ones((8, 8), F32), sq,
                            (((1,), (1,)), ((), ())),
                            preferred_element_type=F32,
                            precision=lax.Precision.HIGHEST)[0:1, :]
    d = rowd2 + cold2 - 2.0 * dot
    INF = jnp.float32(3.0e38)
    BIGI = jnp.int32(N)
    lane = lax.broadcasted_iota(jnp.int32, (_KNN_B, 128), 1)
    ms = [jnp.full((_KNN_B, 128), INF) for _ in range(4)]
    js = [jnp.full((_KNN_B, 128), BIGI) for _ in range(4)]
    for j in range(64):
        e = d[:, j * 128:(j + 1) * 128]
        c = lane + (j * 128)
        for s in range(4):
            cond = (e < ms[s]) | ((e == ms[s]) & (c < js[s]))
            nm = jnp.where(cond, e, ms[s])
            nj = jnp.where(cond, c, js[s])
            e = jnp.where(cond, ms[s], e)
            c = jnp.where(cond, js[s], c)
            ms[s] = nm
            js[s] = nj
    v = jnp.concatenate(ms, axis=1)
    jc = jnp.concatenate(js, axis=1)
    pcnt = jnp.zeros((_KNN_B, 128), jnp.float32)
    flag = jnp.zeros((_KNN_B, 1), jnp.float32)
    for t in range(K):
        if t > 0:
            flag = flag + jnp.max(
                jnp.where(pcnt >= 4.0, 1.0, 0.0), axis=1, keepdims=True)
        vmin = jnp.min(v, axis=1, keepdims=True)
        jm = jnp.min(jnp.where(v <= vmin, jc, BIGI), axis=1, keepdims=True)
        idx_ref[:, t:t + 1] = jm
        v = jnp.where((v <= vmin) & (jc == jm), INF, v)
        pcnt = pcnt + jnp.where(lane == jm % 128, 1.0, 0.0)

    @pl.when(jnp.sum(flag) > 0.0)
    def _():
        dd = d
        cols = lax.broadcasted_iota(jnp.int32, (_KNN_B, N), 1)
        for t in range(K):
            m = jnp.min(dd, axis=1, keepdims=True)
            am = jnp.min(jnp.where(dd <= m, cols, BIGI), axis=1,
                         keepdims=True)
            idx_ref[:, t:t + 1] = am
            dd = jnp.where(cols == am, INF, dd)


def _knn(coord):
    coordp = jnp.pad(coord, ((0, 0), (0, 5)))
    return pl.pallas_call(
        _knn_body,
        grid=(N // _KNN_B,),
        in_specs=[
            pl.BlockSpec((_KNN_B, 8), lambda i: (i, 0)),
            pl.BlockSpec((N, 8), lambda i: (0, 0)),
        ],
        out_specs=pl.BlockSpec((_KNN_B, K), lambda i: (i, 0)),
        out_shape=jax.ShapeDtypeStruct((N, K), jnp.int32),
    )(coordp, coordp)


def _sc_pos(cx_h, cy_h, cz_h, fidx):
    mesh = plsc.VectorSubcoreMesh(core_axis_name="c", subcore_axis_name="s")
    nper = N // 32
    npk = nper * K

    @functools.partial(
        pl.kernel, mesh=mesh,
        out_type=jax.ShapeDtypeStruct((3 * NK,), F32),
        compiler_params=pltpu.CompilerParams(needs_layout_passes=False),
        scratch_types=[
            pltpu.VMEM((N,), F32),
            pltpu.VMEM((N,), F32),
            pltpu.VMEM((N,), F32),
            pltpu.VMEM((npk,), jnp.int32),
            pltpu.VMEM((3 * npk,), F32),
        ],
    )
    def body(cx_hbm, cy_hbm, cz_hbm, fidx_hbm, out_hbm, cx, cy, cz, idxv,
             posb):
        wid = lax.axis_index("s") * 2 + lax.axis_index("c")
        base = wid * nper
        pltpu.sync_copy(cx_hbm, cx)
        pltpu.sync_copy(cy_hbm, cy)
        pltpu.sync_copy(cz_hbm, cz)
        pltpu.sync_copy(fidx_hbm.at[pl.ds(base * K, npk)], idxv)
        lanes = lax.broadcasted_iota(jnp.int32, (16,), 0)

        def step(n, _):
            iv = idxv[pl.ds(n * K, K)]
            ctr_i = jnp.full((16,), base + n, jnp.int32)
            flat = jnp.full((16,), n * K, jnp.int32) + lanes
            for c, cref in ((0, cx), (1, cy), (2, cz)):
                g = plsc.load_gather(cref, [iv])
                ctr = plsc.load_gather(cref, [ctr_i])
                plsc.store_scatter(posb, [flat + c * npk], g - ctr)
            return 0

        lax.fori_loop(0, nper, step, 0)
        for c in range(3):
            pltpu.sync_copy(posb.at[pl.ds(c * npk, npk)],
                            out_hbm.at[pl.ds(c * NK + base * K, npk)])

    return body(cx_h, cy_h, cz_h, fidx)


def _sc_gather8(tab_flat, fidx):
    mesh = plsc.VectorSubcoreMesh(core_axis_name="c", subcore_axis_name="s")
    rper = NK // 32

    @functools.partial(
        pl.kernel, mesh=mesh,
        out_type=jax.ShapeDtypeStruct((G * NK,), F32),
        compiler_params=pltpu.CompilerParams(needs_layout_passes=False),
        scratch_types=[
            pltpu.VMEM((N * G,), F32),
            pltpu.VMEM((rper,), jnp.int32),
            pltpu.VMEM((rper * G,), F32),
        ],
    )
    def body(tab_hbm, fidx_hbm, out_hbm, tabv, idxv, outb):
        wid = lax.axis_index("s") * 2 + lax.axis_index("c")
        base = wid * rper
        pltpu.sync_copy(tab_hbm, tabv)
        pltpu.sync_copy(fidx_hbm.at[pl.ds(base, rper)], idxv)
        lanes = lax.broadcasted_iota(jnp.int32, (16,), 0)

        def step(r, _):
            iv = idxv[pl.ds(r * 16, 16)]
            flat = jnp.full((16,), r * 16, jnp.int32) + lanes
            for g in range(G):
                gg = plsc.load_gather(tabv, [iv + g * N])
                plsc.store_scatter(outb, [flat + g * rper], gg)
            return 0

        lax.fori_loop(0, rper // 16, step, 0)
        for g in range(G):
            pltpu.sync_copy(outb.at[pl.ds(g * rper, rper)],
                            out_hbm.at[pl.ds(g * NK + base, rper)])

    return body(tab_flat, fidx)


def _sc_gather(table, fidx, dcols):
    mesh = plsc.VectorSubcoreMesh(core_axis_name="c", subcore_axis_name="s")
    rper = NK // 32
    chunk = 128
    nch = rper // chunk

    @functools.partial(
        pl.kernel, mesh=mesh,
        out_type=jax.ShapeDtypeStruct((NK, dcols), F32),
        scratch_types=[
            pltpu.VMEM((rper,), jnp.int32),
            pltpu.VMEM((chunk, dcols), F32),
            pltpu.VMEM((chunk, dcols), F32),
            pltpu.SemaphoreType.DMA,
            pltpu.SemaphoreType.DMA,
        ],
    )
    def body(tab_hbm, fidx_hbm, out_hbm, idxv, b0, b1, s0, s1):
        wid = lax.axis_index("s") * 2 + lax.axis_index("c")
        base = wid * rper
        pltpu.sync_copy(fidx_hbm.at[pl.ds(base, rper)], idxv)
        bufs = (b0, b1)
        sems = (s0, s1)
        handles = [None] * nch
        handles[0] = pltpu.async_copy(
            tab_hbm.at[idxv.at[pl.ds(0, chunk)]], b0, s0)
        for c in range(nch):
            if c + 1 < nch:
                handles[c + 1] = pltpu.async_copy(
                    tab_hbm.at[idxv.at[pl.ds((c + 1) * chunk, chunk)]],
                    bufs[(c + 1) % 2], sems[(c + 1) % 2])
            handles[c].wait()
            pltpu.sync_copy(bufs[c % 2],
                            out_hbm.at[pl.ds(base + c * chunk, chunk)])

    return body(table, fidx)


_BM = 512


def _mm_stats_body(x_ref, w_ref, b_ref, y_ref, st_ref):
    y = jnp.dot(x_ref[...], w_ref[...], preferred_element_type=F32) \
        + b_ref[0:1, :]
    y_ref[...] = y
    _acc_stats(st_ref, y, pl.program_id(0) == 0)


def _mm_stats(x, w, b):
    return pl.pallas_call(
        _mm_stats_body,
        grid=(N // _BM,),
        in_specs=[
            pl.BlockSpec((_BM, C), lambda i: (i, 0)),
            pl.BlockSpec((C, C), lambda i: (0, 0)),
            pl.BlockSpec((8, C), lambda i: (0, 0)),
        ],
        out_specs=[
            pl.BlockSpec((_BM, C), lambda i: (i, 0)),
            pl.BlockSpec((8, C), lambda i: (0, 0)),
        ],
        out_shape=[
            jax.ShapeDtypeStruct((N, C), F32),
            jax.ShapeDtypeStruct((8, C), F32),
        ],
    )(x, w, b)


def _fqkv_body(y1_ref, s1_ref, wq_ref, bq_ref, wk_ref, bk_ref, wv_ref, bv_ref,
               yq_ref, yk_ref, v_ref, sq_ref, sk_ref):
    f = jax.nn.relu(_nrm(y1_ref[...], s1_ref, float(N)))
    first = pl.program_id(0) == 0
    yq = jnp.dot(f, wq_ref[...], preferred_element_type=F32) + bq_ref[0:1, :]
    yq_ref[...] = yq
    _acc_stats(sq_ref, yq, first)
    yk = jnp.dot(f, wk_ref[...], preferred_element_type=F32) + bk_ref[0:1, :]
    yk_ref[...] = yk
    _acc_stats(sk_ref, yk, first)
    v_ref[...] = jnp.dot(f, wv_ref[...], preferred_element_type=F32) \
        + bv_ref[0:1, :]


def _fqkv(y1, s1, wq, bq, wk, bk, wv, bv):
    wspec = pl.BlockSpec((C, C), lambda i: (0, 0))
    bspec = pl.BlockSpec((8, C), lambda i: (0, 0))
    blk = pl.BlockSpec((_BM, C), lambda i: (i, 0))
    return pl.pallas_call(
        _fqkv_body,
        grid=(N // _BM,),
        in_specs=[blk, bspec, wspec, bspec, wspec, bspec, wspec, bspec],
        out_specs=[blk, blk, blk, bspec, bspec],
        out_shape=[
            jax.ShapeDtypeStruct((N, C), F32),
            jax.ShapeDtypeStruct((N, C), F32),
            jax.ShapeDtypeStruct((N, C), F32),
            jax.ShapeDtypeStruct((8, C), F32),
            jax.ShapeDtypeStruct((8, C), F32),
        ],
    )(y1, s1, wq, bq, wk, bk, wv, bv)


def _qkproj_body(yq_ref, sq_ref, yk_ref, sk_ref, w1_ref,
                 a_ref, bq_out_ref):
    q = jax.nn.relu(_nrm(yq_ref[...], sq_ref, float(N)))
    kf = jax.nn.relu(_nrm(yk_ref[...], sk_ref, float(N)))
    w1 = w1_ref[...]
    a_ref[...] = lax.dot_general(w1, kf, (((0,), (1,)), ((), ())),
                                 preferred_element_type=F32)
    bq_out_ref[...] = lax.dot_general(w1, q, (((0,), (1,)), ((), ())),
                                      preferred_element_type=F32)


def _qkproj(yq, sq, yk, sk, w1):
    blk = pl.BlockSpec((_BM, C), lambda i: (i, 0))
    oblk = pl.BlockSpec((G, _BM), lambda i: (0, i))
    return pl.pallas_call(
        _qkproj_body,
        grid=(N // _BM,),
        in_specs=[blk, pl.BlockSpec((8, C), lambda i: (0, 0)), blk,
                  pl.BlockSpec((8, C), lambda i: (0, 0)),
                  pl.BlockSpec((C, G), lambda i: (0, 0))],
        out_specs=[oblk, oblk],
        out_shape=[
            jax.ShapeDtypeStruct((G, N), F32),
            jax.ShapeDtypeStruct((G, N), F32),
        ],
    )(yq, sq, yk, sk, w1)


_BP = 2048


def _gp_body(pos_ref, wp1_ref, bp1_ref):
    return lax.dot_general(pos_ref[...], wp1_ref[...],
                           (((0,), (0,)), ((), ())),
                           preferred_element_type=F32) + bp1_ref[0:1, :]


def _gp_stats_body(pos_ref, wp1_ref, bp1_ref, st_ref):
    gp = _gp_body(pos_ref, wp1_ref, bp1_ref)
    _acc_stats(st_ref, gp, pl.program_id(0) == 0)


def _gp_stats(posT, wp1p, bp1):
    return pl.pallas_call(
        _gp_stats_body,
        grid=(NK // _BP,),
        in_specs=[
            pl.BlockSpec((8, _BP), lambda i: (0, i)),
            pl.BlockSpec((8, C), lambda i: (0, 0)),
            pl.BlockSpec((8, C), lambda i: (0, 0)),
        ],
        out_specs=pl.BlockSpec((8, C), lambda i: (0, 0)),
        out_shape=jax.ShapeDtypeStruct((8, C), F32),
    )(posT, wp1p, bp1)


def _nrm_t(y, stats, n):
    m = stats[:, 0:1] / n
    var = stats[:, 1:2] / n - m * m
    return (y - m) * lax.rsqrt(var + EPS)


def _acc_stats_t(ref, y, first):
    @pl.when(first)
    def _():
        ref[...] = jnp.zeros_like(ref)

    ref[:, 0:1] += jnp.sum(y, axis=1, keepdims=True)
    ref[:, 1:2] += jnp.sum(y * y, axis=1, keepdims=True)


def _wpre_body(pos_ref, wp1_ref, bp1_ref, s3_ref, ag_ref, bq_ref,
               wp2_ref, w1_ref, bp2_ref, bw1_ref, wpre_ref, sw_ref):
    gp = _gp_body(pos_ref, wp1_ref, bp1_ref)
    r = jax.nn.relu(_nrm(gp, s3_ref, float(NK)))
    w1 = w1_ref[...]
    wc = jnp.dot(wp2_ref[...], w1, preferred_element_type=F32)
    cw = lax.dot_general(w1, bp2_ref[0:1, :], (((0,), (1,)), ((), ())),
                         preferred_element_type=F32) + bw1_ref[:, 0:1]
    nb = _BP // K
    snb = (lax.broadcasted_iota(jnp.int32, (nb, _BP), 0)
           == lax.broadcasted_iota(jnp.int32, (nb, _BP), 1) // K).astype(F32)
    bq_rep = jnp.dot(bq_ref[...], snb, preferred_element_type=F32)
    rwc = lax.dot_general(wc, r, (((0,), (1,)), ((), ())),
                          preferred_element_type=F32)
    wpre = ag_ref[...] - bq_rep + rwc + cw
    wpre_ref[...] = wpre
    _acc_stats_t(sw_ref, wpre, pl.program_id(0) == 0)


def _wpre(posT, wp1p, bp1, s3, agT, bqT, wp2, w1, bp2, bw1c):
    cspec = pl.BlockSpec((8, C), lambda i: (0, 0))
    return pl.pallas_call(
        _wpre_body,
        grid=(NK // _BP,),
        in_specs=[
            pl.BlockSpec((8, _BP), lambda i: (0, i)),
            cspec, cspec, cspec,
            pl.BlockSpec((G, _BP), lambda i: (0, i)),
            pl.BlockSpec((G, _BP // K), lambda i: (0, i)),
            pl.BlockSpec((C, C), lambda i: (0, 0)),
            pl.BlockSpec((C, G), lambda i: (0, 0)),
            cspec,
            pl.BlockSpec((G, 128), lambda i: (0, 0)),
        ],
        out_specs=[
            pl.BlockSpec((G, _BP), lambda i: (0, i)),
            pl.BlockSpec((G, 128), lambda i: (0, 0)),
        ],
        out_shape=[
            jax.ShapeDtypeStruct((G, NK), F32),
            jax.ShapeDtypeStruct((G, 128), F32),
        ],
    )(posT, wp1p, bp1, s3, agT, bqT, wp2, w1, bp2, bw1c)


def _att_body(wpre_ref, sw_ref, ww2_ref, bw2_ref, pos_ref, wp1_ref, bp1_ref,
              s3_ref, wp2_ref, bp2_ref, vg_ref, wf3_ref, bf3_ref, e8_ref,
              z_ref, sz_ref):
    nb = _BP // K
    s2 = jax.nn.relu(_nrm_t(wpre_ref[...], sw_ref, float(NK)))
    wlin = lax.dot_general(ww2_ref[...], s2, (((0,), (0,)), ((), ())),
                           preferred_element_type=F32) + bw2_ref[:, 0:1]
    e = jnp.exp(wlin)
    snb = (lax.broadcasted_iota(jnp.int32, (nb, _BP), 0)
           == lax.broadcasted_iota(jnp.int32, (nb, _BP), 1) // K).astype(F32)
    denom = lax.dot_general(e, snb, (((1,), (1,)), ((), ())),
                            preferred_element_type=F32)
    w = e * jnp.dot(1.0 / denom, snb, preferred_element_type=F32)
    wt = lax.dot_general(w, e8_ref[...], (((0,), (0,)), ((), ())),
                         preferred_element_type=F32)
    gp = _gp_body(pos_ref, wp1_ref, bp1_ref)
    r = jax.nn.relu(_nrm(gp, s3_ref, float(NK)))
    val = vg_ref[...] + jnp.dot(r, wp2_ref[...], preferred_element_type=F32) \
        + bp2_ref[0:1, :]
    att = jnp.dot(snb, wt * val, preferred_element_type=F32)
    z = jnp.dot(att, wf3_ref[...], preferred_element_type=F32) \
        + bf3_ref[0:1, :]
    z_ref[...] = z
    _acc_stats(sz_ref, z, pl.program_id(0) == 0)


def _att(wpreT, sw, ww2, bw2c, posT, wp1p, bp1, s3, wp2, bp2, vg, wf3, bf3,
         e8):
    cspec = pl.BlockSpec((8, C), lambda i: (0, 0))
    return pl.pallas_call(
        _att_body,
        grid=(NK // _BP,),
        in_specs=[
            pl.BlockSpec((G, _BP), lambda i: (0, i)),
            pl.BlockSpec((G, 128), lambda i: (0, 0)),
            pl.BlockSpec((G, G), lambda i: (0, 0)),
            pl.BlockSpec((G, 128), lambda i: (0, 0)),
            pl.BlockSpec((8, _BP), lambda i: (0, i)),
            cspec, cspec, cspec,
            pl.BlockSpec((C, C), lambda i: (0, 0)),
            cspec,
            pl.BlockSpec((_BP, C), lambda i: (i, 0)),
            pl.BlockSpec((C, C), lambda i: (0, 0)),
            cspec,
            pl.BlockSpec((G, C), lambda i: (0, 0)),
        ],
        out_specs=[
            pl.BlockSpec((_BP // K, C), lambda i: (i, 0)),
            cspec,
        ],
        out_shape=[
            jax.ShapeDtypeStruct((N, C), F32),
            jax.ShapeDtypeStruct((8, C), F32),
        ],
    )(wpreT, sw, ww2, bw2c, posT, wp1p, bp1, s3, wp2, bp2, vg, wf3, bf3,
      e8)


def _resid_body(x_ref, z_ref, sz_ref, out_ref):
    out_ref[...] = jax.nn.relu(x_ref[...] + _nrm(z_ref[...], sz_ref, float(N)))


def _resid(x, z, sz):
    blk = pl.BlockSpec((_BM, C), lambda i: (i, 0))
    return pl.pallas_call(
        _resid_body,
        grid=(N // _BM,),
        in_specs=[blk, blk, pl.BlockSpec((8, C), lambda i: (0, 0))],
        out_specs=blk,
        out_shape=jax.ShapeDtypeStruct((N, C), F32),
    )(x, z, sz)


def kernel(feat, coord, Wq, bq, Wk, bk, Wv, bv, Wp1, bp1, Wp2, bp2,
           Ww1, bw1, Ww2, bw2, Wf1, bf1, Wf3, bf3):
    D = Wq.shape[0]
    idx = _knn(coord)
    fidx = idx.reshape(-1)
    posT = jnp.pad(_sc_pos(coord[:, 0], coord[:, 1], coord[:, 2],
                           fidx).reshape(3, NK), ((0, 5), (0, 0)))

    wp1p = jnp.concatenate([Wp1, jnp.zeros((D, 5, C), F32)], axis=1)
    e8 = (jnp.arange(G)[:, None]
          == (jnp.arange(C)[None, :] // (C // G))).astype(F32)

    def pad_b(b):
        return jnp.broadcast_to(b[None, :], (8, b.shape[0]))

    def pad_c(b):
        return jnp.broadcast_to(b[:, None], (G, 128))

    x = feat
    for i in range(D):
        y1, s1 = _mm_stats(x, Wf1[i], pad_b(bf1[i]))
        yq, yk, v, sq, sk = _fqkv(y1, s1, Wq[i], pad_b(bq[i]), Wk[i],
                                  pad_b(bk[i]), Wv[i], pad_b(bv[i]))
        aT, bqT = _qkproj(yq, sq, yk, sk, Ww1[i])
        agT = _sc_gather8(aT.reshape(-1), fidx).reshape(G, NK)
        vg = _sc_gather(v, fidx, C)
        s3 = _gp_stats(posT, wp1p[i], pad_b(bp1[i]))
        wpreT, sw = _wpre(posT, wp1p[i], pad_b(bp1[i]), s3, agT, bqT,
                          Wp2[i], Ww1[i], pad_b(bp2[i]), pad_c(bw1[i]))
        z, sz = _att(wpreT, sw, Ww2[i], pad_c(bw2[i]), posT, wp1p[i],
                     pad_b(bp1[i]), s3, Wp2[i], pad_b(bp2[i]), vg,
                     Wf3[i], pad_b(bf3[i]), e8)
        x = _resid(x, z, sz)
    return x


# speedup vs baseline: 7.5671x; 1.0301x over previous
import functools

import jax
import jax.numpy as jnp
from jax import lax
from jax.experimental import pallas as pl
from jax.experimental.pallas import tpu as pltpu
from jax.experimental.pallas import tpu_sc as plsc

N = 8192
C = 256
G = 8
K = 16
NK = N * K
EPS = 1e-5
F32 = jnp.float32


def _nrm(y, stats, n):
    m = stats[0:1, :] / n
    var = stats[1:2, :] / n - m * m
    return (y - m) * lax.rsqrt(var + EPS)


def _acc_stats(ref, y, first):
    @pl.when(first)
    def _():
        ref[...] = jnp.zeros_like(ref)

    ref[0:1, :] += jnp.sum(y, axis=0, keepdims=True)
    ref[1:2, :] += jnp.sum(y * y, axis=0, keepdims=True)


_KNN_B = 128


def _knn_body(crd_ref, crdall_ref, idx_ref):
    blk = crd_ref[...]
    allc = crdall_ref[...]
    dot = lax.dot_general(blk, allc, (((1,), (1,)), ((), ())),
                          preferred_element_type=F32)
    rowd2 = jnp.sum(blk * blk, axis=1, keepdims=True)
    sq = allc * allc
    cold2 = lax.dot_general(jnp.ones((8, 8), F32), sq,
                            (((1,), (1,)), ((), ())),
                            preferred_element_type=F32,
                            precision=lax.Precision.HIGHEST)[0:1, :]
    d = rowd2 + cold2 - 2.0 * dot
    INF = jnp.float32(3.0e38)
    BIGI = jnp.int32(N)
    lane = lax.broadcasted_iota(jnp.int32, (_KNN_B, 128), 1)
    ms = [jnp.full((_KNN_B, 128), INF) for _ in range(4)]
    js = [jnp.full((_KNN_B, 128), BIGI) for _ in range(4)]
    for j in range(64):
        e = d[:, j * 128:(j + 1) * 128]
        c = lane + (j * 128)
        for s in range(4):
            cond = (e < ms[s]) | ((e == ms[s]) & (c < js[s]))
            nm = jnp.where(cond, e, ms[s])
            nj = jnp.where(cond, c, js[s])
            e = jnp.where(cond, ms[s], e)
            c = jnp.where(cond, js[s], c)
            ms[s] = nm
            js[s] = nj
    v = jnp.concatenate(ms, axis=1)
    jc = jnp.concatenate(js, axis=1)
    pcnt = jnp.zeros((_KNN_B, 128), jnp.float32)
    flag = jnp.zeros((_KNN_B, 1), jnp.float32)
    for t in range(K):
        if t > 0:
            flag = flag + jnp.max(
                jnp.where(pcnt >= 4.0, 1.0, 0.0), axis=1, keepdims=True)
        vmin = jnp.min(v, axis=1, keepdims=True)
        jm = jnp.min(jnp.where(v <= vmin, jc, BIGI), axis=1, keepdims=True)
        idx_ref[:, t:t + 1] = jm
        v = jnp.where((v <= vmin) & (jc == jm), INF, v)
        pcnt = pcnt + jnp.where(lane == jm % 128, 1.0, 0.0)

    @pl.when(jnp.sum(flag) > 0.0)
    def _():
        dd = d
        cols = lax.broadcasted_iota(jnp.int32, (_KNN_B, N), 1)
        for t in range(K):
            m = jnp.min(dd, axis=1, keepdims=True)
            am = jnp.min(jnp.where(dd <= m, cols, BIGI), axis=1,
                         keepdims=True)
            idx_ref[:, t:t + 1] = am
            dd = jnp.where(cols == am, INF, dd)


def _knn(coord):
    coordp = jnp.pad(coord, ((0, 0), (0, 5)))
    return pl.pallas_call(
        _knn_body,
        grid=(N // _KNN_B,),
        in_specs=[
            pl.BlockSpec((_KNN_B, 8), lambda i: (i, 0)),
            pl.BlockSpec((N, 8), lambda i: (0, 0)),
        ],
        out_specs=pl.BlockSpec((_KNN_B, K), lambda i: (i, 0)),
        out_shape=jax.ShapeDtypeStruct((N, K), jnp.int32),
    )(coordp, coordp)


def _sc_pos(cx_h, cy_h, cz_h, fidx):
    mesh = plsc.VectorSubcoreMesh(core_axis_name="c", subcore_axis_name="s")
    nper = N // 32
    npk = nper * K

    @functools.partial(
        pl.kernel, mesh=mesh,
        out_type=jax.ShapeDtypeStruct((3 * NK,), F32),
        compiler_params=pltpu.CompilerParams(needs_layout_passes=False),
        scratch_types=[
            pltpu.VMEM((N,), F32),
            pltpu.VMEM((N,), F32),
            pltpu.VMEM((N,), F32),
            pltpu.VMEM((npk,), jnp.int32),
            pltpu.VMEM((3 * npk,), F32),
        ],
    )
    def body(cx_hbm, cy_hbm, cz_hbm, fidx_hbm, out_hbm, cx, cy, cz, idxv,
             posb):
        wid = lax.axis_index("s") * 2 + lax.axis_index("c")
        base = wid * nper
        pltpu.sync_copy(cx_hbm, cx)
        pltpu.sync_copy(cy_hbm, cy)
        pltpu.sync_copy(cz_hbm, cz)
        pltpu.sync_copy(fidx_hbm.at[pl.ds(base * K, npk)], idxv)
        lanes = lax.broadcasted_iota(jnp.int32, (16,), 0)

        def step(n, _):
            iv = idxv[pl.ds(n * K, K)]
            ctr_i = jnp.full((16,), base + n, jnp.int32)
            flat = jnp.full((16,), n * K, jnp.int32) + lanes
            for c, cref in ((0, cx), (1, cy), (2, cz)):
                g = plsc.load_gather(cref, [iv])
                ctr = plsc.load_gather(cref, [ctr_i])
                plsc.store_scatter(posb, [flat + c * npk], g - ctr)
            return 0

        lax.fori_loop(0, nper, step, 0)
        for c in range(3):
            pltpu.sync_copy(posb.at[pl.ds(c * npk, npk)],
                            out_hbm.at[pl.ds(c * NK + base * K, npk)])

    return body(cx_h, cy_h, cz_h, fidx)


def _sc_gather8(tab_flat, fidx):
    mesh = plsc.VectorSubcoreMesh(core_axis_name="c", subcore_axis_name="s")
    rper = NK // 32

    @functools.partial(
        pl.kernel, mesh=mesh,
        out_type=jax.ShapeDtypeStruct((G * NK,), F32),
        compiler_params=pltpu.CompilerParams(needs_layout_passes=False),
        scratch_types=[
            pltpu.VMEM((N * G,), F32),
            pltpu.VMEM((rper,), jnp.int32),
            pltpu.VMEM((rper * G,), F32),
        ],
    )
    def body(tab_hbm, fidx_hbm, out_hbm, tabv, idxv, outb):
        wid = lax.axis_index("s") * 2 + lax.axis_index("c")
        base = wid * rper
        pltpu.sync_copy(tab_hbm, tabv)
        pltpu.sync_copy(fidx_hbm.at[pl.ds(base, rper)], idxv)
        lanes = lax.broadcasted_iota(jnp.int32, (16,), 0)

        def step(r, _):
            iv = idxv[pl.ds(r * 16, 16)]
            flat = jnp.full((16,), r * 16, jnp.int32) + lanes
            for g in range(G):
                gg = plsc.load_gather(tabv, [iv + g * N])
                plsc.store_scatter(outb, [flat + g * rper], gg)
            return 0

        lax.fori_loop(0, rper // 16, step, 0)
        for g in range(G):
            pltpu.sync_copy(outb.at[pl.ds(g * rper, rper)],
                            out_hbm.at[pl.ds(g * NK + base, rper)])

    return body(tab_flat, fidx)


def _sc_gather(table, fidx, dcols):
    mesh = plsc.VectorSubcoreMesh(core_axis_name="c", subcore_axis_name="s")
    rper = NK // 32
    chunk = 128
    nch = rper // chunk

    @functools.partial(
        pl.kernel, mesh=mesh,
        out_type=jax.ShapeDtypeStruct((NK, dcols), F32),
        scratch_types=[
            pltpu.VMEM((rper,), jnp.int32),
            pltpu.VMEM((chunk, dcols), F32),
            pltpu.VMEM((chunk, dcols), F32),
            pltpu.SemaphoreType.DMA,
            pltpu.SemaphoreType.DMA,
        ],
    )
    def body(tab_hbm, fidx_hbm, out_hbm, idxv, b0, b1, s0, s1):
        wid = lax.axis_index("s") * 2 + lax.axis_index("c")
        base = wid * rper
        pltpu.sync_copy(fidx_hbm.at[pl.ds(base, rper)], idxv)
        bufs = (b0, b1)
        sems = (s0, s1)
        handles = [None] * nch
        handles[0] = pltpu.async_copy(
            tab_hbm.at[idxv.at[pl.ds(0, chunk)]], b0, s0)
        for c in range(nch):
            if c + 1 < nch:
                handles[c + 1] = pltpu.async_copy(
                    tab_hbm.at[idxv.at[pl.ds((c + 1) * chunk, chunk)]],
                    bufs[(c + 1) % 2], sems[(c + 1) % 2])
            handles[c].wait()
            pltpu.sync_copy(bufs[c % 2],
                            out_hbm.at[pl.ds(base + c * chunk, chunk)])

    return body(table, fidx)


_BM = 512


def _mm_stats_body(x_ref, w_ref, b_ref, y_ref, st_ref):
    y = jnp.dot(x_ref[...], w_ref[...], preferred_element_type=F32) \
        + b_ref[0:1, :]
    y_ref[...] = y
    _acc_stats(st_ref, y, pl.program_id(0) == 0)


def _mm_stats(x, w, b):
    return pl.pallas_call(
        _mm_stats_body,
        grid=(N // _BM,),
        in_specs=[
            pl.BlockSpec((_BM, C), lambda i: (i, 0)),
            pl.BlockSpec((C, C), lambda i: (0, 0)),
            pl.BlockSpec((8, C), lambda i: (0, 0)),
        ],
        out_specs=[
            pl.BlockSpec((_BM, C), lambda i: (i, 0)),
            pl.BlockSpec((8, C), lambda i: (0, 0)),
        ],
        out_shape=[
            jax.ShapeDtypeStruct((N, C), F32),
            jax.ShapeDtypeStruct((8, C), F32),
        ],
    )(x, w, b)


def _fqkv_body(y1_ref, s1_ref, wq_ref, bq_ref, wk_ref, bk_ref, wv_ref, bv_ref,
               yq_ref, yk_ref, v_ref, sq_ref, sk_ref):
    f = jax.nn.relu(_nrm(y1_ref[...], s1_ref, float(N)))
    first = pl.program_id(0) == 0
    yq = jnp.dot(f, wq_ref[...], preferred_element_type=F32) + bq_ref[0:1, :]
    yq_ref[...] = yq
    _acc_stats(sq_ref, yq, first)
    yk = jnp.dot(f, wk_ref[...], preferred_element_type=F32) + bk_ref[0:1, :]
    yk_ref[...] = yk
    _acc_stats(sk_ref, yk, first)
    v_ref[...] = jnp.dot(f, wv_ref[...], preferred_element_type=F32) \
        + bv_ref[0:1, :]


def _fqkv(y1, s1, wq, bq, wk, bk, wv, bv):
    wspec = pl.BlockSpec((C, C), lambda i: (0, 0))
    bspec = pl.BlockSpec((8, C), lambda i: (0, 0))
    blk = pl.BlockSpec((_BM, C), lambda i: (i, 0))
    return pl.pallas_call(
        _fqkv_body,
        grid=(N // _BM,),
        in_specs=[blk, bspec, wspec, bspec, wspec, bspec, wspec, bspec],
        out_specs=[blk, blk, blk, bspec, bspec],
        out_shape=[
            jax.ShapeDtypeStruct((N, C), F32),
            jax.ShapeDtypeStruct((N, C), F32),
            jax.ShapeDtypeStruct((N, C), F32),
            jax.ShapeDtypeStruct((8, C), F32),
            jax.ShapeDtypeStruct((8, C), F32),
        ],
    )(y1, s1, wq, bq, wk, bk, wv, bv)


def _qkproj_body(yq_ref, sq_ref, yk_ref, sk_ref, w1_ref,
                 a_ref, bq_out_ref):
    q = jax.nn.relu(_nrm(yq_ref[...], sq_ref, float(N)))
    kf = jax.nn.relu(_nrm(yk_ref[...], sk_ref, float(N)))
    w1 = w1_ref[...]
    a_ref[...] = lax.dot_general(w1, kf, (((0,), (1,)), ((), ())),
                                 preferred_element_type=F32)
    bq_out_ref[...] = lax.dot_general(w1, q, (((0,), (1,)), ((), ())),
                                      preferred_element_type=F32)


def _qkproj(yq, sq, yk, sk, w1):
    blk = pl.BlockSpec((_BM, C), lambda i: (i, 0))
    oblk = pl.BlockSpec((G, _BM), lambda i: (0, i))
    return pl.pallas_call(
        _qkproj_body,
        grid=(N // _BM,),
        in_specs=[blk, pl.BlockSpec((8, C), lambda i: (0, 0)), blk,
                  pl.BlockSpec((8, C), lambda i: (0, 0)),
                  pl.BlockSpec((C, G), lambda i: (0, 0))],
        out_specs=[oblk, oblk],
        out_shape=[
            jax.ShapeDtypeStruct((G, N), F32),
            jax.ShapeDtypeStruct((G, N), F32),
        ],
    )(yq, sq, yk, sk, w1)


_BP = 2048


def _gp_body(pos_ref, wp1_ref, bp1_ref):
    return lax.dot_general(pos_ref[...], wp1_ref[...],
                           (((0,), (0,)), ((), ())),
                           preferred_element_type=F32) + bp1_ref[0:1, :]


def _gp_stats_body(pos_ref, wp1_ref, bp1_ref, st_ref, p1_ref, p2_ref):
    first = pl.program_id(0) == 0
    last = pl.program_id(0) == (NK // _BP) - 1
    pos = pos_ref[...]

    @pl.when(first)
    def _():
        p1_ref[...] = jnp.zeros_like(p1_ref)
        p2_ref[...] = jnp.zeros_like(p2_ref)

    p1_ref[:, 0:1] += jnp.sum(pos, axis=1, keepdims=True)
    p2_ref[:, 0:8] += lax.dot_general(pos, pos, (((1,), (1,)), ((), ())),
                                      preferred_element_type=F32)

    @pl.when(last)
    def _():
        wp1 = wp1_ref[...]
        bp1 = bp1_ref[0:1, :]
        p1 = p1_ref[:, 0:1]
        t = jnp.dot(p2_ref[:, 0:8], wp1, preferred_element_type=F32)
        diagq = jnp.sum(wp1 * t, axis=0, keepdims=True)
        s1c = jnp.sum(wp1 * p1, axis=0, keepdims=True)
        st_ref[0:1, :] = s1c + float(NK) * bp1
        st_ref[1:2, :] = diagq + 2.0 * bp1 * s1c + float(NK) * bp1 * bp1


def _gp_stats(posT, wp1p, bp1):
    return pl.pallas_call(
        _gp_stats_body,
        grid=(NK // _BP,),
        in_specs=[
            pl.BlockSpec((8, _BP), lambda i: (0, i)),
            pl.BlockSpec((8, C), lambda i: (0, 0)),
            pl.BlockSpec((8, C), lambda i: (0, 0)),
        ],
        out_specs=pl.BlockSpec((8, C), lambda i: (0, 0)),
        out_shape=jax.ShapeDtypeStruct((8, C), F32),
        scratch_shapes=[
            pltpu.VMEM((8, 128), F32),
            pltpu.VMEM((8, 128), F32),
        ],
    )(posT, wp1p, bp1)


def _nrm_t(y, stats, n):
    m = stats[:, 0:1] / n
    var = stats[:, 1:2] / n - m * m
    return (y - m) * lax.rsqrt(var + EPS)


def _acc_stats_t(ref, y, first):
    @pl.when(first)
    def _():
        ref[...] = jnp.zeros_like(ref)

    ref[:, 0:1] += jnp.sum(y, axis=1, keepdims=True)
    ref[:, 1:2] += jnp.sum(y * y, axis=1, keepdims=True)


def _wpre_body(pos_ref, wp1_ref, bp1_ref, s3_ref, ag_ref, bq_ref,
               wp2_ref, w1_ref, bp2_ref, bw1_ref, wpre_ref, sw_ref):
    gp = _gp_body(pos_ref, wp1_ref, bp1_ref)
    r = jax.nn.relu(_nrm(gp, s3_ref, float(NK)))
    w1 = w1_ref[...]
    wc = jnp.dot(wp2_ref[...], w1, preferred_element_type=F32)
    cw = lax.dot_general(w1, bp2_ref[0:1, :], (((0,), (1,)), ((), ())),
                         preferred_element_type=F32) + bw1_ref[:, 0:1]
    nb = _BP // K
    snb = (lax.broadcasted_iota(jnp.int32, (nb, _BP), 0)
           == lax.broadcasted_iota(jnp.int32, (nb, _BP), 1) // K).astype(F32)
    bq_rep = jnp.dot(bq_ref[...], snb, preferred_element_type=F32)
    rwc = lax.dot_general(wc, r, (((0,), (1,)), ((), ())),
                          preferred_element_type=F32)
    wpre = ag_ref[...] - bq_rep + rwc + cw
    wpre_ref[...] = wpre
    _acc_stats_t(sw_ref, wpre, pl.program_id(0) == 0)


def _wpre(posT, wp1p, bp1, s3, agT, bqT, wp2, w1, bp2, bw1c):
    cspec = pl.BlockSpec((8, C), lambda i: (0, 0))
    return pl.pallas_call(
        _wpre_body,
        grid=(NK // _BP,),
        in_specs=[
            pl.BlockSpec((8, _BP), lambda i: (0, i)),
            cspec, cspec, cspec,
            pl.BlockSpec((G, _BP), lambda i: (0, i)),
            pl.BlockSpec((G, _BP // K), lambda i: (0, i)),
            pl.BlockSpec((C, C), lambda i: (0, 0)),
            pl.BlockSpec((C, G), lambda i: (0, 0)),
            cspec,
            pl.BlockSpec((G, 128), lambda i: (0, 0)),
        ],
        out_specs=[
            pl.BlockSpec((G, _BP), lambda i: (0, i)),
            pl.BlockSpec((G, 128), lambda i: (0, 0)),
        ],
        out_shape=[
            jax.ShapeDtypeStruct((G, NK), F32),
            jax.ShapeDtypeStruct((G, 128), F32),
        ],
    )(posT, wp1p, bp1, s3, agT, bqT, wp2, w1, bp2, bw1c)


def _att_body(wpre_ref, sw_ref, ww2_ref, bw2_ref, pos_ref, wp1_ref, bp1_ref,
              s3_ref, wp2_ref, bp2_ref, vg_ref, wf3_ref, bf3_ref, e8_ref,
              z_ref, sz_ref):
    nb = _BP // K
    s2 = jax.nn.relu(_nrm_t(wpre_ref[...], sw_ref, float(NK)))
    wlin = lax.dot_general(ww2_ref[...], s2, (((0,), (0,)), ((), ())),
                           preferred_element_type=F32) + bw2_ref[:, 0:1]
    e = jnp.exp(wlin)
    snb = (lax.broadcasted_iota(jnp.int32, (nb, _BP), 0)
           == lax.broadcasted_iota(jnp.int32, (nb, _BP), 1) // K).astype(F32)
    denom = lax.dot_general(e, snb, (((1,), (1,)), ((), ())),
                            preferred_element_type=F32)
    w = e * jnp.dot(1.0 / denom, snb, preferred_element_type=F32)
    wt = lax.dot_general(w, e8_ref[...], (((0,), (0,)), ((), ())),
                         preferred_element_type=F32)
    gp = _gp_body(pos_ref, wp1_ref, bp1_ref)
    r = jax.nn.relu(_nrm(gp, s3_ref, float(NK)))
    val = vg_ref[...] + jnp.dot(r, wp2_ref[...], preferred_element_type=F32) \
        + bp2_ref[0:1, :]
    att = jnp.dot(snb, wt * val, preferred_element_type=F32)
    z = jnp.dot(att, wf3_ref[...], preferred_element_type=F32) \
        + bf3_ref[0:1, :]
    z_ref[...] = z
    _acc_stats(sz_ref, z, pl.program_id(0) == 0)


def _att(wpreT, sw, ww2, bw2c, posT, wp1p, bp1, s3, wp2, bp2, vg, wf3, bf3,
         e8):
    cspec = pl.BlockSpec((8, C), lambda i: (0, 0))
    return pl.pallas_call(
        _att_body,
        grid=(NK // _BP,),
        in_specs=[
            pl.BlockSpec((G, _BP), lambda i: (0, i)),
            pl.BlockSpec((G, 128), lambda i: (0, 0)),
            pl.BlockSpec((G, G), lambda i: (0, 0)),
            pl.BlockSpec((G, 128), lambda i: (0, 0)),
            pl.BlockSpec((8, _BP), lambda i: (0, i)),
            cspec, cspec, cspec,
            pl.BlockSpec((C, C), lambda i: (0, 0)),
            cspec,
            pl.BlockSpec((_BP, C), lambda i: (i, 0)),
            pl.BlockSpec((C, C), lambda i: (0, 0)),
            cspec,
            pl.BlockSpec((G, C), lambda i: (0, 0)),
        ],
        out_specs=[
            pl.BlockSpec((_BP // K, C), lambda i: (i, 0)),
            cspec,
        ],
        out_shape=[
            jax.ShapeDtypeStruct((N, C), F32),
            jax.ShapeDtypeStruct((8, C), F32),
        ],
    )(wpreT, sw, ww2, bw2c, posT, wp1p, bp1, s3, wp2, bp2, vg, wf3, bf3,
      e8)


def _resid_body(x_ref, z_ref, sz_ref, out_ref):
    out_ref[...] = jax.nn.relu(x_ref[...] + _nrm(z_ref[...], sz_ref, float(N)))


def _resid(x, z, sz):
    blk = pl.BlockSpec((_BM, C), lambda i: (i, 0))
    return pl.pallas_call(
        _resid_body,
        grid=(N // _BM,),
        in_specs=[blk, blk, pl.BlockSpec((8, C), lambda i: (0, 0))],
        out_specs=blk,
        out_shape=jax.ShapeDtypeStruct((N, C), F32),
    )(x, z, sz)


def kernel(feat, coord, Wq, bq, Wk, bk, Wv, bv, Wp1, bp1, Wp2, bp2,
           Ww1, bw1, Ww2, bw2, Wf1, bf1, Wf3, bf3):
    D = Wq.shape[0]
    idx = _knn(coord)
    fidx = idx.reshape(-1)
    posT = jnp.pad(_sc_pos(coord[:, 0], coord[:, 1], coord[:, 2],
                           fidx).reshape(3, NK), ((0, 5), (0, 0)))

    wp1p = jnp.concatenate([Wp1, jnp.zeros((D, 5, C), F32)], axis=1)
    e8 = (jnp.arange(G)[:, None]
          == (jnp.arange(C)[None, :] // (C // G))).astype(F32)

    def pad_b(b):
        return jnp.broadcast_to(b[None, :], (8, b.shape[0]))

    def pad_c(b):
        return jnp.broadcast_to(b[:, None], (G, 128))

    x = feat
    for i in range(D):
        y1, s1 = _mm_stats(x, Wf1[i], pad_b(bf1[i]))
        yq, yk, v, sq, sk = _fqkv(y1, s1, Wq[i], pad_b(bq[i]), Wk[i],
                                  pad_b(bk[i]), Wv[i], pad_b(bv[i]))
        aT, bqT = _qkproj(yq, sq, yk, sk, Ww1[i])
        agT = _sc_gather8(aT.reshape(-1), fidx).reshape(G, NK)
        vg = _sc_gather(v, fidx, C)
        s3 = _gp_stats(posT, wp1p[i], pad_b(bp1[i]))
        wpreT, sw = _wpre(posT, wp1p[i], pad_b(bp1[i]), s3, agT, bqT,
                          Wp2[i], Ww1[i], pad_b(bp2[i]), pad_c(bw1[i]))
        z, sz = _att(wpreT, sw, Ww2[i], pad_c(bw2[i]), posT, wp1p[i],
                     pad_b(bp1[i]), s3, Wp2[i], pad_b(bp2[i]), vg,
                     Wf3[i], pad_b(bf3[i]), e8)
        x = _resid(x, z, sz)
    return x
